# baseline (speedup 1.0000x reference)
import functools

import jax
import jax.numpy as jnp
from jax import lax
from jax.experimental import pallas as pl
from jax.experimental.pallas import tpu as pltpu
from jax.experimental.pallas import tpu_sc as plsc

BATCH_SIZE = 1024
BN_EPS = 1e-5
N_NODES = 10000
F_IN = 128
N_CLASSES = 64
N_EDGES = 320000

E_PER_TILE = N_EDGES // 16
CHUNK = 128
N_PAD = 10240
ROWS_PER_TILE = N_PAD // 16

_GRID = 10
_RB = N_NODES // _GRID


def _prologue_body(x_ref, wln_ref, w1_ref, w2_ref, x0_ref, h_ref):
    x = x_ref[...]
    x0_ref[...] = jnp.dot(x, wln_ref[...], preferred_element_type=jnp.float32)
    h_ref[0] = jnp.dot(x, w1_ref[...], preferred_element_type=jnp.float32)
    h_ref[1] = jnp.dot(x, w2_ref[...], preferred_element_type=jnp.float32)


def _prologue(x, wln, w1, w2):
    return pl.pallas_call(
        _prologue_body,
        grid=(_GRID,),
        in_specs=[
            pl.BlockSpec((_RB, F_IN), lambda i: (i, 0)),
            pl.BlockSpec((F_IN, N_CLASSES), lambda i: (0, 0)),
            pl.BlockSpec((F_IN, N_CLASSES), lambda i: (0, 0)),
            pl.BlockSpec((F_IN, N_CLASSES), lambda i: (0, 0)),
        ],
        out_specs=[
            pl.BlockSpec((_RB, N_CLASSES), lambda i: (i, 0)),
            pl.BlockSpec((2, _RB, N_CLASSES), lambda i: (0, i, 0)),
        ],
        out_shape=[
            jax.ShapeDtypeStruct((N_NODES, N_CLASSES), jnp.float32),
            jax.ShapeDtypeStruct((2, N_NODES, N_CLASSES), jnp.float32),
        ],
        compiler_params=pltpu.CompilerParams(
            dimension_semantics=("arbitrary",)),
    )(x, wln, w1, w2)


QCAP = E_PER_TILE + CHUNK + 16
TRASH0 = E_PER_TILE + CHUNK


_DNUMS = lax.GatherDimensionNumbers(
    offset_dims=(), collapsed_slice_dims=(0,), start_index_map=(0,))


def _splat_last(v):
    return lax.gather(v, jnp.full((16, 1), 15, jnp.int32),
                      dimension_numbers=_DNUMS, slice_sizes=(1,),
                      mode=lax.GatherScatterMode.PROMISE_IN_BOUNDS)
BLK = 2000
N_BLK = E_PER_TILE // BLK


def _sc_body(h_ref, ei1_ref, ei2_ref, w1_ref, w2_ref, out_ref,
             src_q, dst_q, w_q, raw, rows_b, dst_stage, acc, gsem, ssem):
    c = lax.axis_index("c")
    s = lax.axis_index("s")

    _ns = jax.named_scope
    with _ns("zero"):
      def _zrow(r, _):
        for j in range(4):
            rows_b[0, r, pl.ds(j * 16, 16)] = jnp.zeros((16,), jnp.float32)
        return 0
      lax.fori_loop(0, CHUNK, _zrow, 0)
      row0 = s * ROWS_PER_TILE
      def _zcp(k, _):
        pltpu.sync_copy(rows_b.at[0], acc.at[pl.ds(row0 + k * CHUNK, CHUNK)])
        return 0
      lax.fori_loop(0, ROWS_PER_TILE // CHUNK, _zcp, 0)

    base_e = s * E_PER_TILE

    mc = jnp.full((16,), c, jnp.int32) == 0

    def _blk(bi, off):
        boff = base_e + bi * BLK
        d1 = pltpu.async_copy(ei1_ref.at[0, pl.ds(boff, BLK)], raw.at[0],
                              gsem.at[0])
        d2 = pltpu.async_copy(ei1_ref.at[1, pl.ds(boff, BLK)], raw.at[1],
                              gsem.at[0])
        d3 = pltpu.async_copy(w1_ref.at[pl.ds(boff, BLK)], raw.at[2], gsem.at[0])
        d4 = pltpu.async_copy(ei2_ref.at[0, pl.ds(boff, BLK)], raw.at[3],
                              gsem.at[0])
        d5 = pltpu.async_copy(ei2_ref.at[1, pl.ds(boff, BLK)], raw.at[4],
                              gsem.at[0])
        d6 = pltpu.async_copy(w2_ref.at[pl.ds(boff, BLK)], raw.at[5], gsem.at[0])
        for d in (d1, d2, d3, d4, d5, d6):
            d.wait()
        def _gcomp(g, off):
            gb = g * 16
            sv = jnp.where(mc, raw[0, pl.ds(gb, 16)], raw[3, pl.ds(gb, 16)])
            dv = jnp.where(mc, raw[1, pl.ds(gb, 16)], raw[4, pl.ds(gb, 16)])
            wvi = jnp.where(mc, raw[2, pl.ds(gb, 16)], raw[5, pl.ds(gb, 16)])
            wv = plsc.bitcast(wvi, jnp.float32)
            m = ((sv >> 10) == (dv >> 10)) & (wv != 0.0)
            ones = jnp.where(m, jnp.int32(1), jnp.int32(0))
            cs = plsc.cumsum(ones)
            pos = off + cs - ones
            trash = TRASH0 + lax.iota(jnp.int32, 16)
            idx = jnp.where(m, pos, trash)
            plsc.store_scatter(src_q, [idx], sv)
            plsc.store_scatter(dst_q, [idx], dv)
            plsc.store_scatter(w_q, [idx], wv)
            return off + _splat_last(cs)
        return lax.fori_loop(0, BLK // 16, _gcomp, off)
    with _ns("compact"):
        count_v = lax.fori_loop(0, N_BLK, _blk, jnp.zeros((16,), jnp.int32))
    count = count_v[0]

    z_i = jnp.zeros((16,), jnp.int32)
    z_f = jnp.zeros((16,), jnp.float32)
    for j in range(CHUNK // 16):
        src_q[pl.ds(count + j * 16, 16)] = z_i
        dst_q[pl.ds(count + j * 16, 16)] = z_i
        w_q[pl.ds(count + j * 16, 16)] = z_f
    nb = jnp.maximum((count + CHUNK - 1) // CHUNK, 1)

    with _ns("barrier1"):
        plsc.subcore_barrier()

    def _fire_gather(k):
        b = k & 1
        pltpu.async_copy(h_ref.at[c].at[src_q.at[pl.ds(k * CHUNK, CHUNK)]],
                         rows_b.at[b], gsem.at[b])

    def _wait_gather(k):
        b = k & 1
        pltpu.make_async_copy(h_ref.at[c].at[src_q.at[pl.ds(k * CHUNK, CHUNK)]],
                              rows_b.at[b], gsem.at[b]).wait()

    def _fire_scatter(k):
        b = k & 1
        pltpu.async_copy(rows_b.at[b], acc.at[dst_stage.at[b]], ssem.at[b],
                         add=True)

    def _wait_scatter(k):
        b = k & 1
        pltpu.make_async_copy(rows_b.at[b], acc.at[dst_stage.at[b]],
                              ssem.at[b]).wait()

    def _scale(k):
        b = k & 1
        dnums = lax.GatherDimensionNumbers(
            offset_dims=(), collapsed_slice_dims=(0,), start_index_map=(0,))
        def _grp(g, _):
            wv = w_q[pl.ds(k * CHUNK + g * 16, 16)]
            for e in range(16):
                ws = lax.gather(wv, jnp.full((16, 1), e, jnp.int32),
                                dimension_numbers=dnums, slice_sizes=(1,),
                                mode=lax.GatherScatterMode.PROMISE_IN_BOUNDS)
                r = g * 16 + e
                for j in range(N_CLASSES // 16):
                    rows_b[b, r, pl.ds(j * 16, 16)] = (
                        rows_b[b, r, pl.ds(j * 16, 16)] * ws)
            return 0
        lax.fori_loop(0, CHUNK // 16, _grp, 0)

    _fire_gather(0)

    def _chunk(k, _):
        b = k & 1
        @pl.when(k >= 1)
        def _():
            _wait_scatter(k - 1)
        @pl.when(k + 1 < nb)
        def _():
            _fire_gather(k + 1)
        _wait_gather(k)
        _scale(k)
        for j in range(CHUNK // 16):
            dst_stage[b, pl.ds(j * 16, 16)] = dst_q[pl.ds(k * CHUNK + j * 16, 16)]
        _fire_scatter(k)
        return 0
    with _ns("phase2"):
        lax.fori_loop(0, nb, _chunk, 0)
        _wait_scatter(nb - 1)

    with _ns("barrier2"):
        plsc.subcore_barrier()
    with _ns("copyout"):
      def _ocp(k, _):
        r = row0 + k * CHUNK
        pltpu.sync_copy(acc.at[pl.ds(r, CHUNK)], out_ref.at[c, pl.ds(r, CHUNK)])
        return 0
      lax.fori_loop(0, ROWS_PER_TILE // CHUNK, _ocp, 0)


def _sc_convs(h_all, ei1, ei2, w1, w2):
    mesh = plsc.VectorSubcoreMesh(core_axis_name="c", subcore_axis_name="s")
    fn = pl.kernel(
        _sc_body, mesh=mesh,
        out_type=jax.ShapeDtypeStruct((2, N_PAD, N_CLASSES), jnp.float32),
        scratch_types=[
            pltpu.VMEM((QCAP,), jnp.int32),
            pltpu.VMEM((QCAP,), jnp.int32),
            pltpu.VMEM((QCAP,), jnp.float32),
            pltpu.VMEM((6, BLK), jnp.int32),
            pltpu.VMEM((2, CHUNK, N_CLASSES), jnp.float32),
            pltpu.VMEM((2, CHUNK), jnp.int32),
            pltpu.VMEM_SHARED((N_PAD, N_CLASSES), jnp.float32),
            pltpu.SemaphoreType.DMA((2,)),
            pltpu.SemaphoreType.DMA((2,)),
        ],
        compiler_params=pltpu.CompilerParams(use_tc_tiling_on_sc=False,
                                             needs_layout_passes=False),
    )
    return fn(h_all, ei1, ei2, w1, w2)


def _epilogue_body(x0_ref, p_ref, bias_ref, gamma_ref, beta_ref, out_ref):
    inv = 1.0 / jnp.sqrt(1.0 + BN_EPS)
    g = gamma_ref[...] * inv
    off = bias_ref[...] * g + beta_ref[...]
    out_ref[...] = (x0_ref[...] + p_ref[0] + p_ref[1]) * g + off


def _epilogue(x0, p, bias, gamma, beta):
    return pl.pallas_call(
        _epilogue_body,
        grid=(_GRID,),
        in_specs=[
            pl.BlockSpec((_RB, N_CLASSES), lambda i: (i, 0)),
            pl.BlockSpec((2, _RB, N_CLASSES), lambda i: (0, i, 0)),
            pl.BlockSpec((1, N_CLASSES), lambda i: (0, 0)),
            pl.BlockSpec((1, N_CLASSES), lambda i: (0, 0)),
            pl.BlockSpec((1, N_CLASSES), lambda i: (0, 0)),
        ],
        out_specs=pl.BlockSpec((_RB, N_CLASSES), lambda i: (i, 0)),
        out_shape=jax.ShapeDtypeStruct((N_NODES, N_CLASSES), jnp.float32),
        compiler_params=pltpu.CompilerParams(
            dimension_semantics=("arbitrary",)),
    )(x0, p, bias, gamma, beta)


def kernel(features, edge_index, edge_index2, edge_weight, edge_weight2,
           W_ln, b_ln, W1, b1, W2, b2, bn_gamma, bn_beta):
    x0, h_all = _prologue(features, W_ln, W1, W2)

    p = _sc_convs(h_all, edge_index, edge_index2,
                  lax.bitcast_convert_type(edge_weight, jnp.int32),
                  lax.bitcast_convert_type(edge_weight2, jnp.int32))

    bias = (b_ln + b1 + b2).reshape(1, N_CLASSES)
    out = _epilogue(x0, p, bias, bn_gamma.reshape(1, N_CLASSES),
                    bn_beta.reshape(1, N_CLASSES))
    return out


# speedup vs baseline: 1.1033x; 1.1033x over previous
import functools

import jax
import jax.numpy as jnp
from jax import lax
from jax.experimental import pallas as pl
from jax.experimental.pallas import tpu as pltpu
from jax.experimental.pallas import tpu_sc as plsc

BATCH_SIZE = 1024
BN_EPS = 1e-5
N_NODES = 10000
F_IN = 128
N_CLASSES = 64
N_EDGES = 320000

E_PER_TILE = N_EDGES // 16
CHUNK = 128
N_PAD = 10240
ROWS_PER_TILE = N_PAD // 16

_GRID = 10
_RB = N_NODES // _GRID


def _prologue_body(x_ref, wln_ref, w1_ref, w2_ref, x0_ref, h_ref):
    x = x_ref[...]
    x0_ref[...] = jnp.dot(x, wln_ref[...], preferred_element_type=jnp.float32)
    h_ref[0] = jnp.dot(x, w1_ref[...], preferred_element_type=jnp.float32)
    h_ref[1] = jnp.dot(x, w2_ref[...], preferred_element_type=jnp.float32)


def _prologue(x, wln, w1, w2):
    return pl.pallas_call(
        _prologue_body,
        grid=(_GRID,),
        in_specs=[
            pl.BlockSpec((_RB, F_IN), lambda i: (i, 0)),
            pl.BlockSpec((F_IN, N_CLASSES), lambda i: (0, 0)),
            pl.BlockSpec((F_IN, N_CLASSES), lambda i: (0, 0)),
            pl.BlockSpec((F_IN, N_CLASSES), lambda i: (0, 0)),
        ],
        out_specs=[
            pl.BlockSpec((_RB, N_CLASSES), lambda i: (i, 0)),
            pl.BlockSpec((2, _RB, N_CLASSES), lambda i: (0, i, 0)),
        ],
        out_shape=[
            jax.ShapeDtypeStruct((N_NODES, N_CLASSES), jnp.float32),
            jax.ShapeDtypeStruct((2, N_NODES, N_CLASSES), jnp.float32),
        ],
        compiler_params=pltpu.CompilerParams(
            dimension_semantics=("arbitrary",)),
    )(x, wln, w1, w2)


HALF = E_PER_TILE // 2
QCAP = HALF + CHUNK + 16
TRASH0 = HALF + CHUNK
BLK = 2000
N_BLK = HALF // BLK
GUNROLL = 5


_DNUMS = lax.GatherDimensionNumbers(
    offset_dims=(), collapsed_slice_dims=(0,), start_index_map=(0,))


def _splat(v, lane):
    return lax.gather(v, jnp.full((16, 1), lane, jnp.int32),
                      dimension_numbers=_DNUMS, slice_sizes=(1,),
                      mode=lax.GatherScatterMode.PROMISE_IN_BOUNDS)


def _sc_body(h_ref, ei1_ref, ei2_ref, w1_ref, w2_ref, out_ref,
             src_q, dst_q, w_q, raw, rows_b, dst_stage, acc,
             gsem, ssem, rsem):
    c = lax.axis_index("c")
    s = lax.axis_index("s")

    def _zrow(r, _):
        for j in range(4):
            rows_b[0, r, pl.ds(j * 16, 16)] = jnp.zeros((16,), jnp.float32)
        return 0
    lax.fori_loop(0, CHUNK, _zrow, 0)
    row0 = s * ROWS_PER_TILE
    def _zcp(k, _):
        pltpu.sync_copy(rows_b.at[0], acc.at[pl.ds(row0 + k * CHUNK, CHUNK)])
        return 0
    lax.fori_loop(0, ROWS_PER_TILE // CHUNK, _zcp, 0)
    plsc.subcore_barrier()

    base_e = s * E_PER_TILE
    mc = jnp.full((16,), c, jnp.int32) == 0

    def _fire_raw(slot, boff):
        ds = []
        ds.append(pltpu.async_copy(ei1_ref.at[0, pl.ds(boff, BLK)],
                                   raw.at[slot, 0], rsem.at[slot]))
        ds.append(pltpu.async_copy(ei1_ref.at[1, pl.ds(boff, BLK)],
                                   raw.at[slot, 1], rsem.at[slot]))
        ds.append(pltpu.async_copy(w1_ref.at[pl.ds(boff, BLK)],
                                   raw.at[slot, 2], rsem.at[slot]))
        ds.append(pltpu.async_copy(ei2_ref.at[0, pl.ds(boff, BLK)],
                                   raw.at[slot, 3], rsem.at[slot]))
        ds.append(pltpu.async_copy(ei2_ref.at[1, pl.ds(boff, BLK)],
                                   raw.at[slot, 4], rsem.at[slot]))
        ds.append(pltpu.async_copy(w2_ref.at[pl.ds(boff, BLK)],
                                   raw.at[slot, 5], rsem.at[slot]))
        return ds

    def _drain_raw(slot, boff):
        pltpu.make_async_copy(ei1_ref.at[0, pl.ds(boff, BLK)],
                              raw.at[slot, 0], rsem.at[slot]).wait()
        pltpu.make_async_copy(ei1_ref.at[1, pl.ds(boff, BLK)],
                              raw.at[slot, 1], rsem.at[slot]).wait()
        pltpu.make_async_copy(w1_ref.at[pl.ds(boff, BLK)],
                              raw.at[slot, 2], rsem.at[slot]).wait()
        pltpu.make_async_copy(ei2_ref.at[0, pl.ds(boff, BLK)],
                              raw.at[slot, 3], rsem.at[slot]).wait()
        pltpu.make_async_copy(ei2_ref.at[1, pl.ds(boff, BLK)],
                              raw.at[slot, 4], rsem.at[slot]).wait()
        pltpu.make_async_copy(w2_ref.at[pl.ds(boff, BLK)],
                              raw.at[slot, 5], rsem.at[slot]).wait()

    def _compact_block(slot, off_v):
        def _step(t, off_v):
            gb = t * (16 * GUNROLL)
            lanes = []
            for u in range(GUNROLL):
                o = gb + u * 16
                sv = jnp.where(mc, raw[slot, 0, pl.ds(o, 16)],
                               raw[slot, 3, pl.ds(o, 16)])
                dv = jnp.where(mc, raw[slot, 1, pl.ds(o, 16)],
                               raw[slot, 4, pl.ds(o, 16)])
                wvi = jnp.where(mc, raw[slot, 2, pl.ds(o, 16)],
                                raw[slot, 5, pl.ds(o, 16)])
                wv = plsc.bitcast(wvi, jnp.float32)
                m = ((sv >> 10) == (dv >> 10)) & (wv != 0.0)
                ones = jnp.where(m, jnp.int32(1), jnp.int32(0))
                cs = plsc.cumsum(ones)
                lanes.append((sv, dv, wv, m, cs))
            trash = TRASH0 + lax.iota(jnp.int32, 16)
            for sv, dv, wv, m, cs in lanes:
                pos = off_v + cs - jnp.where(m, jnp.int32(1), jnp.int32(0))
                idx = jnp.where(m, pos, trash)
                plsc.store_scatter(src_q, [idx], sv)
                plsc.store_scatter(dst_q, [idx], dv)
                plsc.store_scatter(w_q, [idx], wv)
                off_v = off_v + _splat(cs, 15)
            return off_v
        return lax.fori_loop(0, BLK // (16 * GUNROLL), _step, off_v)

    def _fire_gather(k):
        b = k & 3
        pltpu.async_copy(h_ref.at[c].at[src_q.at[pl.ds(k * CHUNK, CHUNK)]],
                         rows_b.at[b], gsem.at[b])

    def _wait_gather(k):
        b = k & 3
        pltpu.make_async_copy(h_ref.at[c].at[src_q.at[pl.ds(k * CHUNK, CHUNK)]],
                              rows_b.at[b], gsem.at[b]).wait()

    def _fire_scatter(k):
        b = k & 3
        pltpu.async_copy(rows_b.at[b], acc.at[dst_stage.at[b]], ssem.at[b],
                         add=True)

    def _wait_scatter(k):
        b = k & 3
        pltpu.make_async_copy(rows_b.at[b], acc.at[dst_stage.at[b]],
                              ssem.at[b]).wait()

    def _scale(k):
        b = k & 3
        def _grp(g, _):
            wv = w_q[pl.ds(k * CHUNK + g * 16, 16)]
            for e in range(16):
                ws = _splat(wv, e)
                r = g * 16 + e
                for j in range(N_CLASSES // 16):
                    rows_b[b, r, pl.ds(j * 16, 16)] = (
                        rows_b[b, r, pl.ds(j * 16, 16)] * ws)
            return 0
        lax.fori_loop(0, CHUNK // 16, _grp, 0)

    for half in range(2):
        hbase = base_e + half * HALF
        _fire_raw(0, hbase)
        off_v = jnp.zeros((16,), jnp.int32)
        for bi in range(N_BLK):
            _drain_raw(bi & 1, hbase + bi * BLK)
            if bi + 1 < N_BLK:
                _fire_raw(1 - (bi & 1), hbase + (bi + 1) * BLK)
            off_v = _compact_block(bi & 1, off_v)
        count = off_v[0]

        z_i = jnp.zeros((16,), jnp.int32)
        z_f = jnp.zeros((16,), jnp.float32)
        for j in range(CHUNK // 16):
            src_q[pl.ds(count + j * 16, 16)] = z_i
            dst_q[pl.ds(count + j * 16, 16)] = z_i
            w_q[pl.ds(count + j * 16, 16)] = z_f
        nb = jnp.maximum((count + CHUNK - 1) // CHUNK, 1)

        _fire_gather(0)
        @pl.when(nb > 1)
        def _():
            _fire_gather(1)

        def _chunk(k, _):
            b = k & 3
            @pl.when(k >= 2)
            def _():
                _wait_scatter(k - 2)
            @pl.when(k + 2 < nb)
            def _():
                _fire_gather(k + 2)
            _wait_gather(k)
            _scale(k)
            for j in range(CHUNK // 16):
                dst_stage[b, pl.ds(j * 16, 16)] = (
                    dst_q[pl.ds(k * CHUNK + j * 16, 16)])
            _fire_scatter(k)
            return 0
        lax.fori_loop(0, nb, _chunk, 0)
        @pl.when(nb >= 2)
        def _():
            _wait_scatter(nb - 2)
        _wait_scatter(nb - 1)

    plsc.subcore_barrier()
    def _ocp(k, _):
        r = row0 + k * CHUNK
        pltpu.sync_copy(acc.at[pl.ds(r, CHUNK)], out_ref.at[c, pl.ds(r, CHUNK)])
        return 0
    lax.fori_loop(0, ROWS_PER_TILE // CHUNK, _ocp, 0)


def _sc_convs(h_all, ei1, ei2, w1, w2):
    mesh = plsc.VectorSubcoreMesh(core_axis_name="c", subcore_axis_name="s")
    fn = pl.kernel(
        _sc_body, mesh=mesh,
        out_type=jax.ShapeDtypeStruct((2, N_PAD, N_CLASSES), jnp.float32),
        scratch_types=[
            pltpu.VMEM((QCAP,), jnp.int32),
            pltpu.VMEM((QCAP,), jnp.int32),
            pltpu.VMEM((QCAP,), jnp.float32),
            pltpu.VMEM((2, 6, BLK), jnp.int32),
            pltpu.VMEM((4, CHUNK, N_CLASSES), jnp.float32),
            pltpu.VMEM((4, CHUNK), jnp.int32),
            pltpu.VMEM_SHARED((N_PAD, N_CLASSES), jnp.float32),
            pltpu.SemaphoreType.DMA((4,)),
            pltpu.SemaphoreType.DMA((4,)),
            pltpu.SemaphoreType.DMA((2,)),
        ],
        compiler_params=pltpu.CompilerParams(use_tc_tiling_on_sc=False,
                                             needs_layout_passes=False),
    )
    return fn(h_all, ei1, ei2, w1, w2)


def _epilogue_body(x0_ref, p_ref, bias_ref, gamma_ref, beta_ref, out_ref):
    inv = 1.0 / jnp.sqrt(1.0 + BN_EPS)
    g = gamma_ref[...] * inv
    off = bias_ref[...] * g + beta_ref[...]
    out_ref[...] = (x0_ref[...] + p_ref[0] + p_ref[1]) * g + off


def _epilogue(x0, p, bias, gamma, beta):
    return pl.pallas_call(
        _epilogue_body,
        grid=(_GRID,),
        in_specs=[
            pl.BlockSpec((_RB, N_CLASSES), lambda i: (i, 0)),
            pl.BlockSpec((2, _RB, N_CLASSES), lambda i: (0, i, 0)),
            pl.BlockSpec((1, N_CLASSES), lambda i: (0, 0)),
            pl.BlockSpec((1, N_CLASSES), lambda i: (0, 0)),
            pl.BlockSpec((1, N_CLASSES), lambda i: (0, 0)),
        ],
        out_specs=pl.BlockSpec((_RB, N_CLASSES), lambda i: (i, 0)),
        out_shape=jax.ShapeDtypeStruct((N_NODES, N_CLASSES), jnp.float32),
        compiler_params=pltpu.CompilerParams(
            dimension_semantics=("arbitrary",)),
    )(x0, p, bias, gamma, beta)


def kernel(features, edge_index, edge_index2, edge_weight, edge_weight2,
           W_ln, b_ln, W1, b1, W2, b2, bn_gamma, bn_beta):
    x0, h_all = _prologue(features, W_ln, W1, W2)

    p = _sc_convs(h_all, edge_index, edge_index2,
                  lax.bitcast_convert_type(edge_weight, jnp.int32),
                  lax.bitcast_convert_type(edge_weight2, jnp.int32))

    bias = (b_ln + b1 + b2).reshape(1, N_CLASSES)
    out = _epilogue(x0, p, bias, bn_gamma.reshape(1, N_CLASSES),
                    bn_beta.reshape(1, N_CLASSES))
    return out


# speedup vs baseline: 1.1305x; 1.0246x over previous
import functools

import jax
import jax.numpy as jnp
from jax import lax
from jax.experimental import pallas as pl
from jax.experimental.pallas import tpu as pltpu
from jax.experimental.pallas import tpu_sc as plsc

BATCH_SIZE = 1024
BN_EPS = 1e-5
N_NODES = 10000
F_IN = 128
N_CLASSES = 64
N_EDGES = 320000

E_PER_TILE = N_EDGES // 16
CHUNK = 128
N_PAD = 10240
ROWS_PER_TILE = N_PAD // 16

_GRID = 10
_RB = N_NODES // _GRID


def _prologue_body(x_ref, wln_ref, w1_ref, w2_ref, x0_ref, h_ref):
    x = x_ref[...]
    x0_ref[...] = jnp.dot(x, wln_ref[...], preferred_element_type=jnp.float32)
    h_ref[0] = jnp.dot(x, w1_ref[...], preferred_element_type=jnp.float32)
    h_ref[1] = jnp.dot(x, w2_ref[...], preferred_element_type=jnp.float32)


def _prologue(x, wln, w1, w2):
    return pl.pallas_call(
        _prologue_body,
        grid=(_GRID,),
        in_specs=[
            pl.BlockSpec((_RB, F_IN), lambda i: (i, 0)),
            pl.BlockSpec((F_IN, N_CLASSES), lambda i: (0, 0)),
            pl.BlockSpec((F_IN, N_CLASSES), lambda i: (0, 0)),
            pl.BlockSpec((F_IN, N_CLASSES), lambda i: (0, 0)),
        ],
        out_specs=[
            pl.BlockSpec((_RB, N_CLASSES), lambda i: (i, 0)),
            pl.BlockSpec((2, _RB, N_CLASSES), lambda i: (0, i, 0)),
        ],
        out_shape=[
            jax.ShapeDtypeStruct((N_NODES, N_CLASSES), jnp.float32),
            jax.ShapeDtypeStruct((2, N_NODES, N_CLASSES), jnp.float32),
        ],
        compiler_params=pltpu.CompilerParams(
            dimension_semantics=("arbitrary",)),
    )(x, wln, w1, w2)


HALF = E_PER_TILE // 2
QCAP = HALF + CHUNK + 16
TRASH0 = HALF + CHUNK
BLK = 2000
N_BLK = HALF // BLK
GUNROLL = 5


_DNUMS = lax.GatherDimensionNumbers(
    offset_dims=(), collapsed_slice_dims=(0,), start_index_map=(0,))


def _splat(v, lane):
    return lax.gather(v, jnp.full((16, 1), lane, jnp.int32),
                      dimension_numbers=_DNUMS, slice_sizes=(1,),
                      mode=lax.GatherScatterMode.PROMISE_IN_BOUNDS)


def _sc_body(h_ref, ei1_ref, ei2_ref, w1_ref, w2_ref, out_ref,
             src_q, dst_q, w_q, raw, raw_f, rows_b, dst_stage, acc,
             gsem, ssem, rsem):
    c = lax.axis_index("c")
    s = lax.axis_index("s")

    def _zrow(r, _):
        for j in range(4):
            rows_b[0, r, pl.ds(j * 16, 16)] = jnp.zeros((16,), jnp.float32)
        return 0
    lax.fori_loop(0, CHUNK, _zrow, 0)
    row0 = s * ROWS_PER_TILE
    def _zcp(k, _):
        pltpu.sync_copy(rows_b.at[0], acc.at[pl.ds(row0 + k * CHUNK, CHUNK)])
        return 0
    lax.fori_loop(0, ROWS_PER_TILE // CHUNK, _zcp, 0)
    plsc.subcore_barrier()

    base_e = s * E_PER_TILE
    mc = jnp.full((16,), c, jnp.int32) == 0

    def _fire_raw(slot, boff):
        ds = []
        ds.append(pltpu.async_copy(ei1_ref.at[0, pl.ds(boff, BLK)],
                                   raw.at[slot, 0], rsem.at[slot]))
        ds.append(pltpu.async_copy(ei1_ref.at[1, pl.ds(boff, BLK)],
                                   raw.at[slot, 1], rsem.at[slot]))
        ds.append(pltpu.async_copy(w1_ref.at[pl.ds(boff, BLK)],
                                   raw_f.at[slot, 0], rsem.at[slot]))
        ds.append(pltpu.async_copy(ei2_ref.at[0, pl.ds(boff, BLK)],
                                   raw.at[slot, 2], rsem.at[slot]))
        ds.append(pltpu.async_copy(ei2_ref.at[1, pl.ds(boff, BLK)],
                                   raw.at[slot, 3], rsem.at[slot]))
        ds.append(pltpu.async_copy(w2_ref.at[pl.ds(boff, BLK)],
                                   raw_f.at[slot, 1], rsem.at[slot]))
        return ds

    def _drain_raw(slot, boff):
        pltpu.make_async_copy(ei1_ref.at[0, pl.ds(boff, BLK)],
                              raw.at[slot, 0], rsem.at[slot]).wait()
        pltpu.make_async_copy(ei1_ref.at[1, pl.ds(boff, BLK)],
                              raw.at[slot, 1], rsem.at[slot]).wait()
        pltpu.make_async_copy(w1_ref.at[pl.ds(boff, BLK)],
                              raw_f.at[slot, 0], rsem.at[slot]).wait()
        pltpu.make_async_copy(ei2_ref.at[0, pl.ds(boff, BLK)],
                              raw.at[slot, 2], rsem.at[slot]).wait()
        pltpu.make_async_copy(ei2_ref.at[1, pl.ds(boff, BLK)],
                              raw.at[slot, 3], rsem.at[slot]).wait()
        pltpu.make_async_copy(w2_ref.at[pl.ds(boff, BLK)],
                              raw_f.at[slot, 1], rsem.at[slot]).wait()

    def _compact_block(slot, off_v):
        def _step(t, off_v):
            gb = t * (16 * GUNROLL)
            lanes = []
            for u in range(GUNROLL):
                o = gb + u * 16
                sv = jnp.where(mc, raw[slot, 0, pl.ds(o, 16)],
                               raw[slot, 2, pl.ds(o, 16)])
                dv = jnp.where(mc, raw[slot, 1, pl.ds(o, 16)],
                               raw[slot, 3, pl.ds(o, 16)])
                wv = jnp.where(mc, raw_f[slot, 0, pl.ds(o, 16)],
                               raw_f[slot, 1, pl.ds(o, 16)])
                m = ((sv >> 10) == (dv >> 10)) & (wv != 0.0)
                ones = jnp.where(m, jnp.int32(1), jnp.int32(0))
                cs = plsc.cumsum(ones)
                lanes.append((sv, dv, wv, m, cs))
            trash = TRASH0 + lax.iota(jnp.int32, 16)
            for sv, dv, wv, m, cs in lanes:
                pos = off_v + cs - jnp.where(m, jnp.int32(1), jnp.int32(0))
                idx = jnp.where(m, pos, trash)
                plsc.store_scatter(src_q, [idx], sv)
                plsc.store_scatter(dst_q, [idx], dv)
                plsc.store_scatter(w_q, [idx], wv)
                off_v = off_v + _splat(cs, 15)
            return off_v
        return lax.fori_loop(0, BLK // (16 * GUNROLL), _step, off_v)

    def _fire_gather(k):
        b = k & 3
        pltpu.async_copy(h_ref.at[c].at[src_q.at[pl.ds(k * CHUNK, CHUNK)]],
                         rows_b.at[b], gsem.at[b])

    def _wait_gather(k):
        b = k & 3
        pltpu.make_async_copy(h_ref.at[c].at[src_q.at[pl.ds(k * CHUNK, CHUNK)]],
                              rows_b.at[b], gsem.at[b]).wait()

    def _fire_scatter(k):
        b = k & 3
        pltpu.async_copy(rows_b.at[b], acc.at[dst_stage.at[b]], ssem.at[b],
                         add=True)

    def _wait_scatter(k):
        b = k & 3
        pltpu.make_async_copy(rows_b.at[b], acc.at[dst_stage.at[b]],
                              ssem.at[b]).wait()

    def _scale(k):
        b = k & 3
        def _grp(g, _):
            wv = w_q[pl.ds(k * CHUNK + g * 16, 16)]
            for e in range(16):
                ws = _splat(wv, e)
                r = g * 16 + e
                for j in range(N_CLASSES // 16):
                    rows_b[b, r, pl.ds(j * 16, 16)] = (
                        rows_b[b, r, pl.ds(j * 16, 16)] * ws)
            return 0
        lax.fori_loop(0, CHUNK // 16, _grp, 0)

    for half in range(2):
        hbase = base_e + half * HALF
        _fire_raw(0, hbase)
        off_v = jnp.zeros((16,), jnp.int32)
        for bi in range(N_BLK):
            _drain_raw(bi & 1, hbase + bi * BLK)
            if bi + 1 < N_BLK:
                _fire_raw(1 - (bi & 1), hbase + (bi + 1) * BLK)
            off_v = _compact_block(bi & 1, off_v)
        count = off_v[0]

        z_i = jnp.zeros((16,), jnp.int32)
        z_f = jnp.zeros((16,), jnp.float32)
        for j in range(CHUNK // 16):
            src_q[pl.ds(count + j * 16, 16)] = z_i
            dst_q[pl.ds(count + j * 16, 16)] = z_i
            w_q[pl.ds(count + j * 16, 16)] = z_f
        nb = jnp.maximum((count + CHUNK - 1) // CHUNK, 1)

        _fire_gather(0)
        @pl.when(nb > 1)
        def _():
            _fire_gather(1)

        def _chunk(k, _):
            b = k & 3
            @pl.when(k >= 2)
            def _():
                _wait_scatter(k - 2)
            @pl.when(k + 2 < nb)
            def _():
                _fire_gather(k + 2)
            _wait_gather(k)
            _scale(k)
            for j in range(CHUNK // 16):
                dst_stage[b, pl.ds(j * 16, 16)] = (
                    dst_q[pl.ds(k * CHUNK + j * 16, 16)])
            _fire_scatter(k)
            return 0
        lax.fori_loop(0, nb, _chunk, 0)
        @pl.when(nb >= 2)
        def _():
            _wait_scatter(nb - 2)
        _wait_scatter(nb - 1)

    plsc.subcore_barrier()
    def _ocp(k, _):
        r = row0 + k * CHUNK
        pltpu.sync_copy(acc.at[pl.ds(r, CHUNK)], out_ref.at[c, pl.ds(r, CHUNK)])
        return 0
    lax.fori_loop(0, ROWS_PER_TILE // CHUNK, _ocp, 0)


def _sc_convs(h_all, ei1, ei2, w1, w2):
    mesh = plsc.VectorSubcoreMesh(core_axis_name="c", subcore_axis_name="s")
    fn = pl.kernel(
        _sc_body, mesh=mesh,
        out_type=jax.ShapeDtypeStruct((2, N_PAD, N_CLASSES), jnp.float32),
        scratch_types=[
            pltpu.VMEM((QCAP,), jnp.int32),
            pltpu.VMEM((QCAP,), jnp.int32),
            pltpu.VMEM((QCAP,), jnp.float32),
            pltpu.VMEM((2, 4, BLK), jnp.int32),
            pltpu.VMEM((2, 2, BLK), jnp.float32),
            pltpu.VMEM((4, CHUNK, N_CLASSES), jnp.float32),
            pltpu.VMEM((4, CHUNK), jnp.int32),
            pltpu.VMEM_SHARED((N_PAD, N_CLASSES), jnp.float32),
            pltpu.SemaphoreType.DMA((4,)),
            pltpu.SemaphoreType.DMA((4,)),
            pltpu.SemaphoreType.DMA((2,)),
        ],
        compiler_params=pltpu.CompilerParams(use_tc_tiling_on_sc=False,
                                             needs_layout_passes=False),
    )
    return fn(h_all, ei1, ei2, w1, w2)


def _epilogue_body(x0_ref, p_ref, bias_ref, gamma_ref, beta_ref, out_ref):
    inv = 1.0 / jnp.sqrt(1.0 + BN_EPS)
    g = gamma_ref[...] * inv
    off = bias_ref[...] * g + beta_ref[...]
    out_ref[...] = (x0_ref[...] + p_ref[0] + p_ref[1]) * g + off


def _epilogue(x0, p, bias, gamma, beta):
    return pl.pallas_call(
        _epilogue_body,
        grid=(_GRID,),
        in_specs=[
            pl.BlockSpec((_RB, N_CLASSES), lambda i: (i, 0)),
            pl.BlockSpec((2, _RB, N_CLASSES), lambda i: (0, i, 0)),
            pl.BlockSpec((1, N_CLASSES), lambda i: (0, 0)),
            pl.BlockSpec((1, N_CLASSES), lambda i: (0, 0)),
            pl.BlockSpec((1, N_CLASSES), lambda i: (0, 0)),
        ],
        out_specs=pl.BlockSpec((_RB, N_CLASSES), lambda i: (i, 0)),
        out_shape=jax.ShapeDtypeStruct((N_NODES, N_CLASSES), jnp.float32),
        compiler_params=pltpu.CompilerParams(
            dimension_semantics=("arbitrary",)),
    )(x0, p, bias, gamma, beta)


def kernel(features, edge_index, edge_index2, edge_weight, edge_weight2,
           W_ln, b_ln, W1, b1, W2, b2, bn_gamma, bn_beta):
    x0, h_all = _prologue(features, W_ln, W1, W2)

    p = _sc_convs(h_all, edge_index, edge_index2, edge_weight, edge_weight2)

    bias = (b_ln + b1 + b2).reshape(1, N_CLASSES)
    out = _epilogue(x0, p, bias, bn_gamma.reshape(1, N_CLASSES),
                    bn_beta.reshape(1, N_CLASSES))
    return out


# speedup vs baseline: 1.1330x; 1.0022x over previous
import functools

import jax
import jax.numpy as jnp
from jax import lax
from jax.experimental import pallas as pl
from jax.experimental.pallas import tpu as pltpu
from jax.experimental.pallas import tpu_sc as plsc

BATCH_SIZE = 1024
BN_EPS = 1e-5
N_NODES = 10000
F_IN = 128
N_CLASSES = 64
N_EDGES = 320000

E_PER_TILE = N_EDGES // 16
CHUNK = 128
N_PAD = 10240
ROWS_PER_TILE = N_PAD // 16

_GRID = 10
_RB = N_NODES // _GRID


def _prologue_body(x_ref, wln_ref, w1_ref, w2_ref, x0_ref, h_ref):
    x = x_ref[...]
    x0_ref[...] = jnp.dot(x, wln_ref[...], preferred_element_type=jnp.float32)
    h_ref[0] = jnp.dot(x, w1_ref[...], preferred_element_type=jnp.float32)
    h_ref[1] = jnp.dot(x, w2_ref[...], preferred_element_type=jnp.float32)


def _prologue(x, wln, w1, w2):
    return pl.pallas_call(
        _prologue_body,
        grid=(_GRID,),
        in_specs=[
            pl.BlockSpec((_RB, F_IN), lambda i: (i, 0)),
            pl.BlockSpec((F_IN, N_CLASSES), lambda i: (0, 0)),
            pl.BlockSpec((F_IN, N_CLASSES), lambda i: (0, 0)),
            pl.BlockSpec((F_IN, N_CLASSES), lambda i: (0, 0)),
        ],
        out_specs=[
            pl.BlockSpec((_RB, N_CLASSES), lambda i: (i, 0)),
            pl.BlockSpec((2, _RB, N_CLASSES), lambda i: (0, i, 0)),
        ],
        out_shape=[
            jax.ShapeDtypeStruct((N_NODES, N_CLASSES), jnp.float32),
            jax.ShapeDtypeStruct((2, N_NODES, N_CLASSES), jnp.float32),
        ],
        compiler_params=pltpu.CompilerParams(
            dimension_semantics=("arbitrary",)),
    )(x, wln, w1, w2)


HALF = E_PER_TILE // 2
QCAP = HALF + CHUNK + 16
TRASH0 = HALF + CHUNK
BLK = 2000
N_BLK = HALF // BLK
GUNROLL = 5


_DNUMS = lax.GatherDimensionNumbers(
    offset_dims=(), collapsed_slice_dims=(0,), start_index_map=(0,))


def _splat(v, lane):
    return lax.gather(v, jnp.full((16, 1), lane, jnp.int32),
                      dimension_numbers=_DNUMS, slice_sizes=(1,),
                      mode=lax.GatherScatterMode.PROMISE_IN_BOUNDS)


def _sc_body(h_ref, ei1_ref, ei2_ref, w1_ref, w2_ref, out_ref,
             src_q, dst_q, w_q, raw, raw_f, rows_b, dst_stage, acc,
             gsem, ssem, rsem):
    c = lax.axis_index("c")
    s = lax.axis_index("s")

    def _zrow(r, _):
        for j in range(4):
            rows_b[0, r, pl.ds(j * 16, 16)] = jnp.zeros((16,), jnp.float32)
        return 0
    lax.fori_loop(0, CHUNK, _zrow, 0)
    row0 = s * ROWS_PER_TILE
    def _zcp(k, _):
        pltpu.sync_copy(rows_b.at[0], acc.at[pl.ds(row0 + k * CHUNK, CHUNK)])
        return 0
    lax.fori_loop(0, ROWS_PER_TILE // CHUNK, _zcp, 0)
    plsc.subcore_barrier()

    base_e = s * E_PER_TILE
    mc = jnp.full((16,), c, jnp.int32) == 0

    def _fire_raw(slot, boff):
        ds = []
        ds.append(pltpu.async_copy(ei1_ref.at[0, pl.ds(boff, BLK)],
                                   raw.at[slot, 0], rsem.at[slot]))
        ds.append(pltpu.async_copy(ei1_ref.at[1, pl.ds(boff, BLK)],
                                   raw.at[slot, 1], rsem.at[slot]))
        ds.append(pltpu.async_copy(w1_ref.at[pl.ds(boff, BLK)],
                                   raw_f.at[slot, 0], rsem.at[slot]))
        ds.append(pltpu.async_copy(ei2_ref.at[0, pl.ds(boff, BLK)],
                                   raw.at[slot, 2], rsem.at[slot]))
        ds.append(pltpu.async_copy(ei2_ref.at[1, pl.ds(boff, BLK)],
                                   raw.at[slot, 3], rsem.at[slot]))
        ds.append(pltpu.async_copy(w2_ref.at[pl.ds(boff, BLK)],
                                   raw_f.at[slot, 1], rsem.at[slot]))
        return ds

    def _drain_raw(slot, boff):
        pltpu.make_async_copy(ei1_ref.at[0, pl.ds(boff, BLK)],
                              raw.at[slot, 0], rsem.at[slot]).wait()
        pltpu.make_async_copy(ei1_ref.at[1, pl.ds(boff, BLK)],
                              raw.at[slot, 1], rsem.at[slot]).wait()
        pltpu.make_async_copy(w1_ref.at[pl.ds(boff, BLK)],
                              raw_f.at[slot, 0], rsem.at[slot]).wait()
        pltpu.make_async_copy(ei2_ref.at[0, pl.ds(boff, BLK)],
                              raw.at[slot, 2], rsem.at[slot]).wait()
        pltpu.make_async_copy(ei2_ref.at[1, pl.ds(boff, BLK)],
                              raw.at[slot, 3], rsem.at[slot]).wait()
        pltpu.make_async_copy(w2_ref.at[pl.ds(boff, BLK)],
                              raw_f.at[slot, 1], rsem.at[slot]).wait()

    def _compact_block(slot, off_v):
        def _step(t, off_v):
            gb = t * (16 * GUNROLL)
            lanes = []
            for u in range(GUNROLL):
                o = gb + u * 16
                sv = jnp.where(mc, raw[slot, 0, pl.ds(o, 16)],
                               raw[slot, 2, pl.ds(o, 16)])
                dv = jnp.where(mc, raw[slot, 1, pl.ds(o, 16)],
                               raw[slot, 3, pl.ds(o, 16)])
                wv = jnp.where(mc, raw_f[slot, 0, pl.ds(o, 16)],
                               raw_f[slot, 1, pl.ds(o, 16)])
                m = ((sv >> 10) == (dv >> 10)) & (wv != 0.0)
                ones = jnp.where(m, jnp.int32(1), jnp.int32(0))
                cs = plsc.cumsum(ones)
                lanes.append((sv, dv, wv, m, cs))
            trash = TRASH0 + lax.iota(jnp.int32, 16)
            for sv, dv, wv, m, cs in lanes:
                pos = off_v + cs - jnp.where(m, jnp.int32(1), jnp.int32(0))
                idx = jnp.where(m, pos, trash)
                plsc.store_scatter(src_q, [idx], sv)
                plsc.store_scatter(dst_q, [idx], dv)
                plsc.store_scatter(w_q, [idx], wv)
                off_v = off_v + _splat(cs, 15)
            return off_v
        return lax.fori_loop(0, BLK // (16 * GUNROLL), _step, off_v)

    def _fire_gather(k):
        b = k & 3
        pltpu.async_copy(h_ref.at[c].at[src_q.at[pl.ds(k * CHUNK, CHUNK)]],
                         rows_b.at[b], gsem.at[b])

    def _wait_gather(k):
        b = k & 3
        pltpu.make_async_copy(h_ref.at[c].at[src_q.at[pl.ds(k * CHUNK, CHUNK)]],
                              rows_b.at[b], gsem.at[b]).wait()

    def _fire_scatter(k):
        b = k & 3
        pltpu.async_copy(rows_b.at[b], acc.at[dst_stage.at[b]], ssem.at[b],
                         add=True)

    def _wait_scatter(k):
        b = k & 3
        pltpu.make_async_copy(rows_b.at[b], acc.at[dst_stage.at[b]],
                              ssem.at[b]).wait()

    def _scale(k):
        b = k & 3
        def _grp(g, _):
            wv = w_q[pl.ds(k * CHUNK + g * 16, 16)]
            for e in range(16):
                ws = _splat(wv, e)
                r = g * 16 + e
                for j in range(N_CLASSES // 16):
                    rows_b[b, r, pl.ds(j * 16, 16)] = (
                        rows_b[b, r, pl.ds(j * 16, 16)] * ws)
            return 0
        lax.fori_loop(0, CHUNK // 16, _grp, 0)

    for half in range(2):
      with jax.named_scope("compact"):
        hbase = base_e + half * HALF
        _fire_raw(0, hbase)
        off_v = jnp.zeros((16,), jnp.int32)
        for bi in range(N_BLK):
            _drain_raw(bi & 1, hbase + bi * BLK)
            if bi + 1 < N_BLK:
                _fire_raw(1 - (bi & 1), hbase + (bi + 1) * BLK)
            off_v = _compact_block(bi & 1, off_v)
        count = off_v[0]

        z_i = jnp.zeros((16,), jnp.int32)
        z_f = jnp.zeros((16,), jnp.float32)
        for j in range(CHUNK // 16):
            src_q[pl.ds(count + j * 16, 16)] = z_i
            dst_q[pl.ds(count + j * 16, 16)] = z_i
            w_q[pl.ds(count + j * 16, 16)] = z_f
        nb = jnp.maximum((count + CHUNK - 1) // CHUNK, 1)

      with jax.named_scope("phase2"):
        _fire_gather(0)
        @pl.when(nb > 1)
        def _():
            _fire_gather(1)

        def _chunk(k, _):
            b = k & 3
            @pl.when(k >= 2)
            def _():
                _wait_scatter(k - 2)
            @pl.when(k + 2 < nb)
            def _():
                _fire_gather(k + 2)
            _wait_gather(k)
            _scale(k)
            for j in range(CHUNK // 16):
                dst_stage[b, pl.ds(j * 16, 16)] = (
                    dst_q[pl.ds(k * CHUNK + j * 16, 16)])
            _fire_scatter(k)
            return 0
        lax.fori_loop(0, nb, _chunk, 0)
        @pl.when(nb >= 2)
        def _():
            _wait_scatter(nb - 2)
        _wait_scatter(nb - 1)

    plsc.subcore_barrier()
    def _ocp(k, _):
        r = row0 + k * CHUNK
        pltpu.sync_copy(acc.at[pl.ds(r, CHUNK)], out_ref.at[c, pl.ds(r, CHUNK)])
        return 0
    lax.fori_loop(0, ROWS_PER_TILE // CHUNK, _ocp, 0)


def _sc_convs(h_all, ei1, ei2, w1, w2):
    mesh = plsc.VectorSubcoreMesh(core_axis_name="c", subcore_axis_name="s")
    fn = pl.kernel(
        _sc_body, mesh=mesh,
        out_type=jax.ShapeDtypeStruct((2, N_PAD, N_CLASSES), jnp.float32),
        scratch_types=[
            pltpu.VMEM((QCAP,), jnp.int32),
            pltpu.VMEM((QCAP,), jnp.int32),
            pltpu.VMEM((QCAP,), jnp.float32),
            pltpu.VMEM((2, 4, BLK), jnp.int32),
            pltpu.VMEM((2, 2, BLK), jnp.float32),
            pltpu.VMEM((4, CHUNK, N_CLASSES), jnp.float32),
            pltpu.VMEM((4, CHUNK), jnp.int32),
            pltpu.VMEM_SHARED((N_PAD, N_CLASSES), jnp.float32),
            pltpu.SemaphoreType.DMA((4,)),
            pltpu.SemaphoreType.DMA((4,)),
            pltpu.SemaphoreType.DMA((2,)),
        ],
        compiler_params=pltpu.CompilerParams(use_tc_tiling_on_sc=False,
                                             needs_layout_passes=False),
    )
    return fn(h_all, ei1, ei2, w1, w2)


def _epilogue_body(x0_ref, p_ref, bias_ref, gamma_ref, beta_ref, out_ref):
    inv = 1.0 / jnp.sqrt(1.0 + BN_EPS)
    g = gamma_ref[...] * inv
    off = bias_ref[...] * g + beta_ref[...]
    out_ref[...] = (x0_ref[...] + p_ref[0] + p_ref[1]) * g + off


def _epilogue(x0, p, bias, gamma, beta):
    return pl.pallas_call(
        _epilogue_body,
        grid=(_GRID,),
        in_specs=[
            pl.BlockSpec((_RB, N_CLASSES), lambda i: (i, 0)),
            pl.BlockSpec((2, _RB, N_CLASSES), lambda i: (0, i, 0)),
            pl.BlockSpec((1, N_CLASSES), lambda i: (0, 0)),
            pl.BlockSpec((1, N_CLASSES), lambda i: (0, 0)),
            pl.BlockSpec((1, N_CLASSES), lambda i: (0, 0)),
        ],
        out_specs=pl.BlockSpec((_RB, N_CLASSES), lambda i: (i, 0)),
        out_shape=jax.ShapeDtypeStruct((N_NODES, N_CLASSES), jnp.float32),
        compiler_params=pltpu.CompilerParams(
            dimension_semantics=("arbitrary",)),
    )(x0, p, bias, gamma, beta)


def kernel(features, edge_index, edge_index2, edge_weight, edge_weight2,
           W_ln, b_ln, W1, b1, W2, b2, bn_gamma, bn_beta):
    x0, h_all = _prologue(features, W_ln, W1, W2)

    p = _sc_convs(h_all, edge_index, edge_index2, edge_weight, edge_weight2)

    bias = (b_ln + b1 + b2).reshape(1, N_CLASSES)
    out = _epilogue(x0, p, bias, bn_gamma.reshape(1, N_CLASSES),
                    bn_beta.reshape(1, N_CLASSES))
    return out
